# Initial kernel scaffold; baseline (speedup 1.0000x reference)
import jax
import jax.numpy as jnp
from jax.experimental import pallas as pl


def kernel(x, edge_index, W1_self, b1_self, W1_neigh, b1_neigh, W2_self, b2_self, W2_neigh, b2_neigh, W_out, b_out, W_lin1, b_lin1):
    raise NotImplementedError("write your pallas kernel here")


# speedup vs baseline: 9.4439x; 9.4439x over previous
import functools

import jax
import jax.numpy as jnp
from jax import lax
from jax.experimental import pallas as pl
from jax.experimental.pallas import tpu as pltpu
from jax.experimental.pallas import tpu_sc as plsc

N = 10000
E = 320000
F = 128
H = 32
OUT = 128

NC = 2
NS = 16
NW = NC * NS
BATCH = 128
K = 8
TPW = 80
ROWS = NW * TPW
EPAD = ROWS * BATCH
NPAD = N + 16
NCNT = 10240
BLK = 1000
GRID = N // BLK


def _mm1_body(x_ref, wn_ref, ws_ref, bs_ref, xn_ref, xs_ref):
    xb = x_ref[:]
    xn_ref[:] = jnp.dot(xb, wn_ref[:], preferred_element_type=jnp.float32)
    xs_ref[:] = (jnp.dot(xb, ws_ref[:], preferred_element_type=jnp.float32)
                 + bs_ref[0:1, :])


def _tc1(x, wnT, wsT, bs):
    return pl.pallas_call(
        _mm1_body,
        grid=(GRID,),
        in_specs=[
            pl.BlockSpec((BLK, F), lambda i: (i, 0)),
            pl.BlockSpec((F, H), lambda i: (0, 0)),
            pl.BlockSpec((F, H), lambda i: (0, 0)),
            pl.BlockSpec((8, H), lambda i: (0, 0)),
        ],
        out_specs=[
            pl.BlockSpec((BLK, H), lambda i: (i, 0)),
            pl.BlockSpec((BLK, H), lambda i: (i, 0)),
        ],
        out_shape=[
            jax.ShapeDtypeStruct((N, H), jnp.float32),
            jax.ShapeDtypeStruct((N, H), jnp.float32),
        ],
    )(x, wnT, wsT, bs)


def _mm2_body(agg_ref, cnt_ref, xs1_ref, bn_ref, w2n_ref, w2s_ref, b2s_ref,
              xn2_ref, xs2_ref):
    asum = agg_ref[0] + agg_ref[1]
    csum = cnt_ref[0] + cnt_ref[1]
    mean = asum / jnp.maximum(csum, 1.0)
    h1 = jnp.maximum(mean + bn_ref[0:1, :] + xs1_ref[:], 0.0)
    xn2_ref[:] = jnp.dot(h1, w2n_ref[:], preferred_element_type=jnp.float32)
    xs2_ref[:] = (jnp.dot(h1, w2s_ref[:], preferred_element_type=jnp.float32)
                  + b2s_ref[0:1, :])


def _tc2(agg, cnt3, xs1, bn, w2nT, w2sT, b2s):
    return pl.pallas_call(
        _mm2_body,
        grid=(GRID,),
        in_specs=[
            pl.BlockSpec((2, BLK, H), lambda i: (0, i, 0)),
            pl.BlockSpec((2, BLK, 1), lambda i: (0, i, 0)),
            pl.BlockSpec((BLK, H), lambda i: (i, 0)),
            pl.BlockSpec((8, H), lambda i: (0, 0)),
            pl.BlockSpec((H, H), lambda i: (0, 0)),
            pl.BlockSpec((H, H), lambda i: (0, 0)),
            pl.BlockSpec((8, H), lambda i: (0, 0)),
        ],
        out_specs=[
            pl.BlockSpec((BLK, H), lambda i: (i, 0)),
            pl.BlockSpec((BLK, H), lambda i: (i, 0)),
        ],
        out_shape=[
            jax.ShapeDtypeStruct((N, H), jnp.float32),
            jax.ShapeDtypeStruct((N, H), jnp.float32),
        ],
    )(agg, cnt3, xs1, bn, w2nT, w2sT, b2s)


def _mm3_body(agg_ref, cnt_ref, xs2_ref, bn_ref, wout_ref, bout_ref,
              h2_ref, out_ref):
    asum = agg_ref[0] + agg_ref[1]
    csum = cnt_ref[0] + cnt_ref[1]
    mean = asum / jnp.maximum(csum, 1.0)
    h2 = jnp.maximum(mean + bn_ref[0:1, :] + xs2_ref[:], 0.0)
    h2_ref[:] = h2
    out_ref[:] = (jnp.dot(h2, wout_ref[:], preferred_element_type=jnp.float32)
                  + bout_ref[0:1, :])


def _tc3(agg, cnt3, xs2, bn, woutT, bout):
    return pl.pallas_call(
        _mm3_body,
        grid=(GRID,),
        in_specs=[
            pl.BlockSpec((2, BLK, H), lambda i: (0, i, 0)),
            pl.BlockSpec((2, BLK, 1), lambda i: (0, i, 0)),
            pl.BlockSpec((BLK, H), lambda i: (i, 0)),
            pl.BlockSpec((8, H), lambda i: (0, 0)),
            pl.BlockSpec((H, OUT), lambda i: (0, 0)),
            pl.BlockSpec((8, OUT), lambda i: (0, 0)),
        ],
        out_specs=[
            pl.BlockSpec((BLK, H), lambda i: (i, 0)),
            pl.BlockSpec((BLK, OUT), lambda i: (i, 0)),
        ],
        out_shape=[
            jax.ShapeDtypeStruct((N, H), jnp.float32),
            jax.ShapeDtypeStruct((N, OUT), jnp.float32),
        ],
    )(agg, cnt3, xs2, bn, woutT, bout)


def _sc_agg_call(xn, src2d, dst2d, zagg, zcnt, with_cnt):
    mesh = plsc.VectorSubcoreMesh(core_axis_name="c", subcore_axis_name="s")
    if with_cnt:
        out_type = (jax.ShapeDtypeStruct((2, N, H), jnp.float32),
                    jax.ShapeDtypeStruct((2, NCNT), jnp.float32))
    else:
        out_type = jax.ShapeDtypeStruct((2, N, H), jnp.float32)

    scratch = [
        pltpu.VMEM((TPW, BATCH), jnp.int32),
        pltpu.VMEM((TPW, BATCH), jnp.int32),
        pltpu.VMEM((K, BATCH, H), jnp.float32),
        pltpu.VMEM((BATCH,), jnp.float32),
        pltpu.VMEM_SHARED((NPAD, H), jnp.float32),
        pltpu.VMEM_SHARED((NCNT,), jnp.float32),
        pltpu.SemaphoreType.DMA,
    ]

    def body(xn_hbm, src_hbm, dst_hbm, zagg_hbm, zcnt_hbm, *rest):
        if with_cnt:
            agg_out, cnt_out = rest[0], rest[1]
            rest = rest[2:]
        else:
            agg_out = rest[0]
            rest = rest[1:]
        idx_src, idx_dst, rows, ones_v, sh_agg, sh_cnt, sem = rest

        c = lax.axis_index("c")
        s = lax.axis_index("s")
        wid = c * NS + s
        base = pl.multiple_of(wid * TPW, TPW)

        pltpu.sync_copy(src_hbm.at[pl.ds(base, TPW)], idx_src)
        pltpu.sync_copy(dst_hbm.at[pl.ds(base, TPW)], idx_dst)
        if with_cnt:
            for i in range(BATCH // 16):
                ones_v[pl.ds(i * 16, 16)] = jnp.ones((16,), jnp.float32)

        @pl.when(s == 0)
        def _zero():
            pltpu.sync_copy(zagg_hbm, sh_agg)
            if with_cnt:
                pltpu.sync_copy(zcnt_hbm, sh_cnt)

        plsc.subcore_barrier()

        def group(g, carry):
            cps = []
            for i in range(K):
                b = g * K + i
                cps.append(pltpu.async_copy(
                    xn_hbm.at[idx_src.at[b]], rows.at[i], sem))
            for cp in cps:
                cp.wait()
            for i in range(K):
                b = g * K + i
                pltpu.sync_copy(rows.at[i], sh_agg.at[idx_dst.at[b]],
                                add=True)
                if with_cnt:
                    pltpu.sync_copy(ones_v, sh_cnt.at[idx_dst.at[b]],
                                    add=True)
            return carry

        lax.fori_loop(0, TPW // K, group, 0)
        plsc.subcore_barrier()

        @pl.when(s < 10)
        def _copy_out():
            r0 = pl.multiple_of(s * 1000, 1000)
            pltpu.sync_copy(sh_agg.at[pl.ds(r0, 1000)],
                            agg_out.at[c].at[pl.ds(r0, 1000)])
            if with_cnt:
                rc = pl.multiple_of(s * 1024, 1024)
                pltpu.sync_copy(sh_cnt.at[pl.ds(rc, 1024)],
                                cnt_out.at[c].at[pl.ds(rc, 1024)])

    fn = pl.kernel(body, out_type=out_type, mesh=mesh, scratch_types=scratch,
                   compiler_params=pltpu.CompilerParams(
                       use_tc_tiling_on_sc=False))
    return fn(xn, src2d, dst2d, zagg, zcnt)


def kernel(x, edge_index, W1_self, b1_self, W1_neigh, b1_neigh,
           W2_self, b2_self, W2_neigh, b2_neigh, W_out, b_out,
           W_lin1, b_lin1):
    src = edge_index[0]
    dst = edge_index[1]
    pad = EPAD - E
    src2d = jnp.concatenate(
        [src, jnp.zeros((pad,), jnp.int32)]).reshape(ROWS, BATCH)
    dst2d = jnp.concatenate(
        [dst, jnp.full((pad,), N, jnp.int32)]).reshape(ROWS, BATCH)
    zagg = jnp.zeros((NPAD, H), jnp.float32)
    zcnt = jnp.zeros((NCNT,), jnp.float32)

    def rep(b):
        return jnp.tile(b.reshape(1, -1), (8, 1))

    xn1, xs1 = _tc1(x, W1_neigh.T, W1_self.T, rep(b1_self))
    agg1, cnt = _sc_agg_call(xn1, src2d, dst2d, zagg, zcnt, with_cnt=True)
    cnt3 = cnt[:, :N].reshape(2, N, 1)
    xn2, xs2 = _tc2(agg1, cnt3, xs1, rep(b1_neigh), W2_neigh.T, W2_self.T,
                    rep(b2_self))

    agg2 = _sc_agg_call(xn2, src2d, dst2d, zagg, zcnt, with_cnt=False)
    h2, out = _tc3(agg2, cnt3, xs2, rep(b2_neigh), W_out.T, rep(b_out))
    return (h2, out)


# speedup vs baseline: 10.8942x; 1.1536x over previous
import functools

import jax
import jax.numpy as jnp
from jax import lax
from jax.experimental import pallas as pl
from jax.experimental.pallas import tpu as pltpu
from jax.experimental.pallas import tpu_sc as plsc

N = 10000
E = 320000
F = 128
H = 32
OUT = 128

NC = 2
NS = 16
NW = NC * NS
BATCH = 128
K = 8
TPW = 80
ROWS = NW * TPW
EPAD = ROWS * BATCH
NPAD = N + 16
NCNT = 10240
BLK = 1000
GRID = N // BLK


def _mm1_body(x_ref, wn_ref, ws_ref, bs_ref, xn_ref, xs_ref):
    xb = x_ref[:]
    xn_ref[:] = jnp.dot(xb, wn_ref[:], preferred_element_type=jnp.float32)
    xs_ref[:] = (jnp.dot(xb, ws_ref[:], preferred_element_type=jnp.float32)
                 + bs_ref[0:1, :])


def _tc1(x, wnT, wsT, bs):
    return pl.pallas_call(
        _mm1_body,
        grid=(GRID,),
        in_specs=[
            pl.BlockSpec((BLK, F), lambda i: (i, 0)),
            pl.BlockSpec((F, H), lambda i: (0, 0)),
            pl.BlockSpec((F, H), lambda i: (0, 0)),
            pl.BlockSpec((8, H), lambda i: (0, 0)),
        ],
        out_specs=[
            pl.BlockSpec((BLK, H), lambda i: (i, 0)),
            pl.BlockSpec((BLK, H), lambda i: (i, 0)),
        ],
        out_shape=[
            jax.ShapeDtypeStruct((N, H), jnp.float32),
            jax.ShapeDtypeStruct((N, H), jnp.float32),
        ],
    )(x, wnT, wsT, bs)


def _mm2_body(agg_ref, cnt_ref, xs1_ref, bn_ref, w2n_ref, w2s_ref, b2s_ref,
              xn2_ref, xs2_ref):
    asum = agg_ref[0] + agg_ref[1]
    csum = cnt_ref[0] + cnt_ref[1]
    mean = asum / jnp.maximum(csum, 1.0)
    h1 = jnp.maximum(mean + bn_ref[0:1, :] + xs1_ref[:], 0.0)
    xn2_ref[:] = jnp.dot(h1, w2n_ref[:], preferred_element_type=jnp.float32)
    xs2_ref[:] = (jnp.dot(h1, w2s_ref[:], preferred_element_type=jnp.float32)
                  + b2s_ref[0:1, :])


def _tc2(agg, cnt3, xs1, bn, w2nT, w2sT, b2s):
    return pl.pallas_call(
        _mm2_body,
        grid=(GRID,),
        in_specs=[
            pl.BlockSpec((2, BLK, H), lambda i: (0, i, 0)),
            pl.BlockSpec((2, BLK, 1), lambda i: (0, i, 0)),
            pl.BlockSpec((BLK, H), lambda i: (i, 0)),
            pl.BlockSpec((8, H), lambda i: (0, 0)),
            pl.BlockSpec((H, H), lambda i: (0, 0)),
            pl.BlockSpec((H, H), lambda i: (0, 0)),
            pl.BlockSpec((8, H), lambda i: (0, 0)),
        ],
        out_specs=[
            pl.BlockSpec((BLK, H), lambda i: (i, 0)),
            pl.BlockSpec((BLK, H), lambda i: (i, 0)),
        ],
        out_shape=[
            jax.ShapeDtypeStruct((N, H), jnp.float32),
            jax.ShapeDtypeStruct((N, H), jnp.float32),
        ],
    )(agg, cnt3, xs1, bn, w2nT, w2sT, b2s)


def _mm3_body(agg_ref, cnt_ref, xs2_ref, bn_ref, wout_ref, bout_ref,
              h2_ref, out_ref):
    asum = agg_ref[0] + agg_ref[1]
    csum = cnt_ref[0] + cnt_ref[1]
    mean = asum / jnp.maximum(csum, 1.0)
    h2 = jnp.maximum(mean + bn_ref[0:1, :] + xs2_ref[:], 0.0)
    h2_ref[:] = h2
    out_ref[:] = (jnp.dot(h2, wout_ref[:], preferred_element_type=jnp.float32)
                  + bout_ref[0:1, :])


def _tc3(agg, cnt3, xs2, bn, woutT, bout):
    return pl.pallas_call(
        _mm3_body,
        grid=(GRID,),
        in_specs=[
            pl.BlockSpec((2, BLK, H), lambda i: (0, i, 0)),
            pl.BlockSpec((2, BLK, 1), lambda i: (0, i, 0)),
            pl.BlockSpec((BLK, H), lambda i: (i, 0)),
            pl.BlockSpec((8, H), lambda i: (0, 0)),
            pl.BlockSpec((H, OUT), lambda i: (0, 0)),
            pl.BlockSpec((8, OUT), lambda i: (0, 0)),
        ],
        out_specs=[
            pl.BlockSpec((BLK, H), lambda i: (i, 0)),
            pl.BlockSpec((BLK, OUT), lambda i: (i, 0)),
        ],
        out_shape=[
            jax.ShapeDtypeStruct((N, H), jnp.float32),
            jax.ShapeDtypeStruct((N, OUT), jnp.float32),
        ],
    )(agg, cnt3, xs2, bn, woutT, bout)


def _sc_agg_call(xn, src2d, dst2d, zagg, zcnt, with_cnt):
    mesh = plsc.VectorSubcoreMesh(core_axis_name="c", subcore_axis_name="s")
    if with_cnt:
        out_type = (jax.ShapeDtypeStruct((2, N, H), jnp.float32),
                    jax.ShapeDtypeStruct((2, NCNT), jnp.float32))
    else:
        out_type = jax.ShapeDtypeStruct((2, N, H), jnp.float32)

    k = 4 if with_cnt else 8
    ngroups = TPW // k

    scratch = [
        pltpu.VMEM((TPW, BATCH), jnp.int32),
        pltpu.VMEM((TPW, BATCH), jnp.int32),
        pltpu.VMEM((k, BATCH, H), jnp.float32),
        pltpu.VMEM((BATCH,), jnp.float32),
        pltpu.VMEM_SHARED((NPAD, H), jnp.float32),
        pltpu.VMEM_SHARED((NCNT,), jnp.float32),
    ] + [pltpu.SemaphoreType.DMA] * k

    def body(xn_hbm, src_hbm, dst_hbm, zagg_hbm, zcnt_hbm, *rest):
        if with_cnt:
            agg_out, cnt_out = rest[0], rest[1]
            rest = rest[2:]
        else:
            agg_out = rest[0]
            rest = rest[1:]
        idx_src, idx_dst, rows, ones_v = rest[:4]
        sh_agg, sh_cnt = rest[4], rest[5]
        sems = rest[6:6 + k]

        c = lax.axis_index("c")
        s = lax.axis_index("s")
        wid = c * NS + s
        base = pl.multiple_of(wid * TPW, TPW)

        pltpu.sync_copy(src_hbm.at[pl.ds(base, TPW)], idx_src)
        pltpu.sync_copy(dst_hbm.at[pl.ds(base, TPW)], idx_dst)
        if with_cnt:
            for i in range(BATCH // 16):
                ones_v[pl.ds(i * 16, 16)] = jnp.ones((16,), jnp.float32)

        rz = pl.multiple_of(s * (NPAD // NS), NPAD // NS)
        pltpu.sync_copy(zagg_hbm.at[pl.ds(rz, NPAD // NS)],
                        sh_agg.at[pl.ds(rz, NPAD // NS)])
        if with_cnt:
            rc = pl.multiple_of(s * (NCNT // NS), NCNT // NS)
            pltpu.sync_copy(zcnt_hbm.at[pl.ds(rc, NCNT // NS)],
                            sh_cnt.at[pl.ds(rc, NCNT // NS)])

        plsc.subcore_barrier()

        for i in range(k):
            pltpu.async_copy(xn_hbm.at[idx_src.at[i]], rows.at[i], sems[i])

        def group(g, carry):
            for i in range(k):
                b = g * k + i
                pltpu.make_async_copy(xn_hbm.at[idx_src.at[b]], rows.at[i],
                                      sems[i]).wait()
                pltpu.sync_copy(rows.at[i], sh_agg.at[idx_dst.at[b]],
                                add=True)
                if with_cnt:
                    pltpu.sync_copy(ones_v, sh_cnt.at[idx_dst.at[b]],
                                    add=True)

                @pl.when(g < ngroups - 1)
                def _rearm():
                    pltpu.async_copy(xn_hbm.at[idx_src.at[b + k]],
                                     rows.at[i], sems[i])
            return carry

        lax.fori_loop(0, ngroups, group, 0)
        plsc.subcore_barrier()

        ro = pl.multiple_of(s * (N // NS), N // NS)
        pltpu.sync_copy(sh_agg.at[pl.ds(ro, N // NS)],
                        agg_out.at[c].at[pl.ds(ro, N // NS)])
        if with_cnt:
            rc2 = pl.multiple_of(s * (NCNT // NS), NCNT // NS)
            pltpu.sync_copy(sh_cnt.at[pl.ds(rc2, NCNT // NS)],
                            cnt_out.at[c].at[pl.ds(rc2, NCNT // NS)])

    fn = pl.kernel(body, out_type=out_type, mesh=mesh, scratch_types=scratch,
                   compiler_params=pltpu.CompilerParams(
                       use_tc_tiling_on_sc=False))
    return fn(xn, src2d, dst2d, zagg, zcnt)


def kernel(x, edge_index, W1_self, b1_self, W1_neigh, b1_neigh,
           W2_self, b2_self, W2_neigh, b2_neigh, W_out, b_out,
           W_lin1, b_lin1):
    src = edge_index[0]
    dst = edge_index[1]
    pad = EPAD - E
    src2d = jnp.concatenate(
        [src, jnp.zeros((pad,), jnp.int32)]).reshape(ROWS, BATCH)
    dst2d = jnp.concatenate(
        [dst, jnp.full((pad,), N, jnp.int32)]).reshape(ROWS, BATCH)
    zagg = jnp.zeros((NPAD, H), jnp.float32)
    zcnt = jnp.zeros((NCNT,), jnp.float32)

    def rep(b):
        return jnp.tile(b.reshape(1, -1), (8, 1))

    xn1, xs1 = _tc1(x, W1_neigh.T, W1_self.T, rep(b1_self))
    agg1, cnt = _sc_agg_call(xn1, src2d, dst2d, zagg, zcnt, with_cnt=True)
    cnt3 = cnt[:, :N].reshape(2, N, 1)
    xn2, xs2 = _tc2(agg1, cnt3, xs1, rep(b1_neigh), W2_neigh.T, W2_self.T,
                    rep(b2_self))

    agg2 = _sc_agg_call(xn2, src2d, dst2d, zagg, zcnt, with_cnt=False)
    h2, out = _tc3(agg2, cnt3, xs2, rep(b2_neigh), W_out.T, rep(b_out))
    return (h2, out)


# speedup vs baseline: 16.1965x; 1.4867x over previous
import functools

import jax
import jax.numpy as jnp
from jax import lax
from jax.experimental import pallas as pl
from jax.experimental.pallas import tpu as pltpu
from jax.experimental.pallas import tpu_sc as plsc

N = 10000
E = 320000
F = 128
H = 32
OUT = 128

NC = 2
NS = 16
NW = NC * NS
BATCH = 128
K = 8
TPW = 80
ROWS = NW * TPW
EPAD = ROWS * BATCH
NPAD = N + 16
NCNT = 10240
BLK = 2000
GRID = N // BLK
EDT = jnp.bfloat16


def _mm1_body(x_ref, wn_ref, ws_ref, bs_ref, xn_ref, xs_ref):
    xb = x_ref[:]
    xn_ref[:] = jnp.dot(
        xb, wn_ref[:], preferred_element_type=jnp.float32).astype(EDT)
    xs_ref[:] = (jnp.dot(xb, ws_ref[:], preferred_element_type=jnp.float32)
                 + bs_ref[0:1, :])


def _tc1(x, wnT, wsT, bs):
    return pl.pallas_call(
        _mm1_body,
        grid=(GRID,),
        in_specs=[
            pl.BlockSpec((BLK, F), lambda i: (i, 0)),
            pl.BlockSpec((F, H), lambda i: (0, 0)),
            pl.BlockSpec((F, H), lambda i: (0, 0)),
            pl.BlockSpec((8, H), lambda i: (0, 0)),
        ],
        out_specs=[
            pl.BlockSpec((BLK, H), lambda i: (i, 0)),
            pl.BlockSpec((BLK, H), lambda i: (i, 0)),
        ],
        out_shape=[
            jax.ShapeDtypeStruct((N, H), EDT),
            jax.ShapeDtypeStruct((N, H), jnp.float32),
        ],
    )(x, wnT, wsT, bs)


def _mm2_body(agg_ref, cnt_ref, xs1_ref, bn_ref, w2n_ref, w2s_ref, b2s_ref,
              xn2_ref, xs2_ref):
    asum = (agg_ref[0].astype(jnp.float32)
            + agg_ref[1].astype(jnp.float32))
    csum = cnt_ref[0] + cnt_ref[1]
    mean = asum / jnp.maximum(csum, 1.0)
    h1 = jnp.maximum(mean + bn_ref[0:1, :] + xs1_ref[:], 0.0)
    xn2_ref[:] = jnp.dot(
        h1, w2n_ref[:], preferred_element_type=jnp.float32).astype(EDT)
    xs2_ref[:] = (jnp.dot(h1, w2s_ref[:], preferred_element_type=jnp.float32)
                  + b2s_ref[0:1, :])


def _tc2(agg, cnt3, xs1, bn, w2nT, w2sT, b2s):
    return pl.pallas_call(
        _mm2_body,
        grid=(GRID,),
        in_specs=[
            pl.BlockSpec((2, BLK, H), lambda i: (0, i, 0)),
            pl.BlockSpec((2, BLK, 1), lambda i: (0, i, 0)),
            pl.BlockSpec((BLK, H), lambda i: (i, 0)),
            pl.BlockSpec((8, H), lambda i: (0, 0)),
            pl.BlockSpec((H, H), lambda i: (0, 0)),
            pl.BlockSpec((H, H), lambda i: (0, 0)),
            pl.BlockSpec((8, H), lambda i: (0, 0)),
        ],
        out_specs=[
            pl.BlockSpec((BLK, H), lambda i: (i, 0)),
            pl.BlockSpec((BLK, H), lambda i: (i, 0)),
        ],
        out_shape=[
            jax.ShapeDtypeStruct((N, H), EDT),
            jax.ShapeDtypeStruct((N, H), jnp.float32),
        ],
    )(agg, cnt3, xs1, bn, w2nT, w2sT, b2s)


def _mm3_body(agg_ref, cnt_ref, xs2_ref, bn_ref, wout_ref, bout_ref,
              h2_ref, out_ref):
    asum = (agg_ref[0].astype(jnp.float32)
            + agg_ref[1].astype(jnp.float32))
    csum = cnt_ref[0] + cnt_ref[1]
    mean = asum / jnp.maximum(csum, 1.0)
    h2 = jnp.maximum(mean + bn_ref[0:1, :] + xs2_ref[:], 0.0)
    h2_ref[:] = h2
    out_ref[:] = (jnp.dot(h2, wout_ref[:], preferred_element_type=jnp.float32)
                  + bout_ref[0:1, :])


def _tc3(agg, cnt3, xs2, bn, woutT, bout):
    return pl.pallas_call(
        _mm3_body,
        grid=(GRID,),
        in_specs=[
            pl.BlockSpec((2, BLK, H), lambda i: (0, i, 0)),
            pl.BlockSpec((2, BLK, 1), lambda i: (0, i, 0)),
            pl.BlockSpec((BLK, H), lambda i: (i, 0)),
            pl.BlockSpec((8, H), lambda i: (0, 0)),
            pl.BlockSpec((H, OUT), lambda i: (0, 0)),
            pl.BlockSpec((8, OUT), lambda i: (0, 0)),
        ],
        out_specs=[
            pl.BlockSpec((BLK, H), lambda i: (i, 0)),
            pl.BlockSpec((BLK, OUT), lambda i: (i, 0)),
        ],
        out_shape=[
            jax.ShapeDtypeStruct((N, H), jnp.float32),
            jax.ShapeDtypeStruct((N, OUT), jnp.float32),
        ],
    )(agg, cnt3, xs2, bn, woutT, bout)


def _sc_agg_call(xn, src2d, dst2d, zagg, zcnt, with_cnt):
    mesh = plsc.VectorSubcoreMesh(core_axis_name="c", subcore_axis_name="s")
    if with_cnt:
        out_type = (jax.ShapeDtypeStruct((2, N, H), EDT),
                    jax.ShapeDtypeStruct((2, NCNT), jnp.float32))
    else:
        out_type = jax.ShapeDtypeStruct((2, N, H), EDT)

    k = 4 if with_cnt else 8
    ngroups = TPW // k

    scratch = [
        pltpu.VMEM((TPW, BATCH), jnp.int32),
        pltpu.VMEM((TPW, BATCH), jnp.int32),
        pltpu.VMEM((k, BATCH, H), EDT),
        pltpu.VMEM((BATCH,), jnp.float32),
        pltpu.VMEM_SHARED((NPAD, H), EDT),
        pltpu.VMEM_SHARED((NCNT,), jnp.float32),
    ] + [pltpu.SemaphoreType.DMA] * k

    def body(xn_hbm, src_hbm, dst_hbm, zagg_hbm, zcnt_hbm, *rest):
        if with_cnt:
            agg_out, cnt_out = rest[0], rest[1]
            rest = rest[2:]
        else:
            agg_out = rest[0]
            rest = rest[1:]
        idx_src, idx_dst, rows, ones_v = rest[:4]
        sh_agg, sh_cnt = rest[4], rest[5]
        sems = rest[6:6 + k]

        c = lax.axis_index("c")
        s = lax.axis_index("s")
        wid = c * NS + s
        base = pl.multiple_of(wid * TPW, TPW)

        pltpu.sync_copy(src_hbm.at[pl.ds(base, TPW)], idx_src)
        pltpu.sync_copy(dst_hbm.at[pl.ds(base, TPW)], idx_dst)
        if with_cnt:
            for i in range(BATCH // 16):
                ones_v[pl.ds(i * 16, 16)] = jnp.ones((16,), jnp.float32)

        rz = pl.multiple_of(s * (NPAD // NS), NPAD // NS)
        pltpu.sync_copy(zagg_hbm.at[pl.ds(rz, NPAD // NS)],
                        sh_agg.at[pl.ds(rz, NPAD // NS)])
        if with_cnt:
            rc = pl.multiple_of(s * (NCNT // NS), NCNT // NS)
            pltpu.sync_copy(zcnt_hbm.at[pl.ds(rc, NCNT // NS)],
                            sh_cnt.at[pl.ds(rc, NCNT // NS)])

        plsc.subcore_barrier()

        for i in range(k):
            pltpu.async_copy(xn_hbm.at[idx_src.at[i]], rows.at[i], sems[i])

        def group(g, carry):
            for i in range(k):
                b = g * k + i
                pltpu.make_async_copy(xn_hbm.at[idx_src.at[b]], rows.at[i],
                                      sems[i]).wait()
                pltpu.sync_copy(rows.at[i], sh_agg.at[idx_dst.at[b]],
                                add=True)
                if with_cnt:
                    pltpu.sync_copy(ones_v, sh_cnt.at[idx_dst.at[b]],
                                    add=True)

                @pl.when(g < ngroups - 1)
                def _rearm():
                    pltpu.async_copy(xn_hbm.at[idx_src.at[b + k]],
                                     rows.at[i], sems[i])
            return carry

        lax.fori_loop(0, ngroups, group, 0)
        plsc.subcore_barrier()

        ro = pl.multiple_of(s * (N // NS), N // NS)
        pltpu.sync_copy(sh_agg.at[pl.ds(ro, N // NS)],
                        agg_out.at[c].at[pl.ds(ro, N // NS)])
        if with_cnt:
            rc2 = pl.multiple_of(s * (NCNT // NS), NCNT // NS)
            pltpu.sync_copy(sh_cnt.at[pl.ds(rc2, NCNT // NS)],
                            cnt_out.at[c].at[pl.ds(rc2, NCNT // NS)])

    fn = pl.kernel(body, out_type=out_type, mesh=mesh, scratch_types=scratch,
                   compiler_params=pltpu.CompilerParams(
                       use_tc_tiling_on_sc=False))
    return fn(xn, src2d, dst2d, zagg, zcnt)


def kernel(x, edge_index, W1_self, b1_self, W1_neigh, b1_neigh,
           W2_self, b2_self, W2_neigh, b2_neigh, W_out, b_out,
           W_lin1, b_lin1):
    src = edge_index[0]
    dst = edge_index[1]
    pad = EPAD - E
    src2d = jnp.concatenate(
        [src, jnp.zeros((pad,), jnp.int32)]).reshape(ROWS, BATCH)
    dst2d = jnp.concatenate(
        [dst, jnp.full((pad,), N, jnp.int32)]).reshape(ROWS, BATCH)
    zagg = jnp.zeros((NPAD, H), EDT)
    zcnt = jnp.zeros((NCNT,), jnp.float32)

    def rep(b):
        return jnp.tile(b.reshape(1, -1), (8, 1))

    xn1, xs1 = _tc1(x, W1_neigh.T, W1_self.T, rep(b1_self))
    agg1, cnt = _sc_agg_call(xn1, src2d, dst2d, zagg, zcnt, with_cnt=True)
    cnt3 = cnt[:, :N].reshape(2, N, 1)
    xn2, xs2 = _tc2(agg1, cnt3, xs1, rep(b1_neigh), W2_neigh.T, W2_self.T,
                    rep(b2_self))

    agg2 = _sc_agg_call(xn2, src2d, dst2d, zagg, zcnt, with_cnt=False)
    h2, out = _tc3(agg2, cnt3, xs2, rep(b2_neigh), W_out.T, rep(b_out))
    return (h2, out)


# speedup vs baseline: 17.0763x; 1.0543x over previous
import functools

import jax
import jax.numpy as jnp
from jax import lax
from jax.experimental import pallas as pl
from jax.experimental.pallas import tpu as pltpu
from jax.experimental.pallas import tpu_sc as plsc

N = 10000
E = 320000
F = 128
H = 32
OUT = 128

NC = 2
NS = 16
NW = NC * NS
BATCH = 128
EROWS = E // BATCH
RPT = EROWS // NW
REM = EROWS - RPT * NW
TPW = 80
NPAD = N + 16
NCNT = 10240
BLK = 2000
GRID = N // BLK
EDT = jnp.bfloat16


def _mm1_body(x_ref, wn_ref, ws_ref, bs_ref, xn_ref, xs_ref):
    xb = x_ref[:]
    xn_ref[:] = jnp.dot(
        xb, wn_ref[:], preferred_element_type=jnp.float32).astype(EDT)
    xs_ref[:] = (jnp.dot(xb, ws_ref[:], preferred_element_type=jnp.float32)
                 + bs_ref[0:1, :])


def _tc1(x, wnT, wsT, bs):
    return pl.pallas_call(
        _mm1_body,
        grid=(GRID,),
        in_specs=[
            pl.BlockSpec((BLK, F), lambda i: (i, 0)),
            pl.BlockSpec((F, H), lambda i: (0, 0)),
            pl.BlockSpec((F, H), lambda i: (0, 0)),
            pl.BlockSpec((8, H), lambda i: (0, 0)),
        ],
        out_specs=[
            pl.BlockSpec((BLK, H), lambda i: (i, 0)),
            pl.BlockSpec((BLK, H), lambda i: (i, 0)),
        ],
        out_shape=[
            jax.ShapeDtypeStruct((N, H), EDT),
            jax.ShapeDtypeStruct((N, H), jnp.float32),
        ],
    )(x, wnT, wsT, bs)


def _mm2_body(agg_ref, cnt_ref, xs1_ref, bn_ref, w2n_ref, w2s_ref, b2s_ref,
              xn2_ref, xs2_ref):
    asum = (agg_ref[0].astype(jnp.float32)
            + agg_ref[1].astype(jnp.float32))
    csum = cnt_ref[0] + cnt_ref[1]
    mean = asum / jnp.maximum(csum, 1.0)
    h1 = jnp.maximum(mean + bn_ref[0:1, :] + xs1_ref[:], 0.0)
    xn2_ref[:] = jnp.dot(
        h1, w2n_ref[:], preferred_element_type=jnp.float32).astype(EDT)
    xs2_ref[:] = (jnp.dot(h1, w2s_ref[:], preferred_element_type=jnp.float32)
                  + b2s_ref[0:1, :])


def _tc2(agg, cnt3, xs1, bn, w2nT, w2sT, b2s):
    return pl.pallas_call(
        _mm2_body,
        grid=(GRID,),
        in_specs=[
            pl.BlockSpec((2, BLK, H), lambda i: (0, i, 0)),
            pl.BlockSpec((2, BLK, 1), lambda i: (0, i, 0)),
            pl.BlockSpec((BLK, H), lambda i: (i, 0)),
            pl.BlockSpec((8, H), lambda i: (0, 0)),
            pl.BlockSpec((H, H), lambda i: (0, 0)),
            pl.BlockSpec((H, H), lambda i: (0, 0)),
            pl.BlockSpec((8, H), lambda i: (0, 0)),
        ],
        out_specs=[
            pl.BlockSpec((BLK, H), lambda i: (i, 0)),
            pl.BlockSpec((BLK, H), lambda i: (i, 0)),
        ],
        out_shape=[
            jax.ShapeDtypeStruct((N, H), EDT),
            jax.ShapeDtypeStruct((N, H), jnp.float32),
        ],
    )(agg, cnt3, xs1, bn, w2nT, w2sT, b2s)


def _mm3_body(agg_ref, cnt_ref, xs2_ref, bn_ref, wout_ref, bout_ref,
              h2_ref, out_ref):
    asum = (agg_ref[0].astype(jnp.float32)
            + agg_ref[1].astype(jnp.float32))
    csum = cnt_ref[0] + cnt_ref[1]
    mean = asum / jnp.maximum(csum, 1.0)
    h2 = jnp.maximum(mean + bn_ref[0:1, :] + xs2_ref[:], 0.0)
    h2_ref[:] = h2
    out_ref[:] = (jnp.dot(h2, wout_ref[:], preferred_element_type=jnp.float32)
                  + bout_ref[0:1, :])


def _tc3(agg, cnt3, xs2, bn, woutT, bout):
    return pl.pallas_call(
        _mm3_body,
        grid=(GRID,),
        in_specs=[
            pl.BlockSpec((2, BLK, H), lambda i: (0, i, 0)),
            pl.BlockSpec((2, BLK, 1), lambda i: (0, i, 0)),
            pl.BlockSpec((BLK, H), lambda i: (i, 0)),
            pl.BlockSpec((8, H), lambda i: (0, 0)),
            pl.BlockSpec((H, OUT), lambda i: (0, 0)),
            pl.BlockSpec((8, OUT), lambda i: (0, 0)),
        ],
        out_specs=[
            pl.BlockSpec((BLK, H), lambda i: (i, 0)),
            pl.BlockSpec((BLK, OUT), lambda i: (i, 0)),
        ],
        out_shape=[
            jax.ShapeDtypeStruct((N, H), jnp.float32),
            jax.ShapeDtypeStruct((N, OUT), jnp.float32),
        ],
    )(agg, cnt3, xs2, bn, woutT, bout)


def _sc_agg_call(xn, e3, zagg, zcnt, with_cnt):
    mesh = plsc.VectorSubcoreMesh(core_axis_name="c", subcore_axis_name="s")
    if with_cnt:
        out_type = (jax.ShapeDtypeStruct((2, N, H), EDT),
                    jax.ShapeDtypeStruct((2, NCNT), jnp.float32))
    else:
        out_type = jax.ShapeDtypeStruct((2, N, H), EDT)

    k = 4 if with_cnt else 8
    ngroups = TPW // k

    scratch = [
        pltpu.VMEM((TPW, BATCH), jnp.int32),
        pltpu.VMEM((TPW, BATCH), jnp.int32),
        pltpu.VMEM((k, BATCH, H), EDT),
        pltpu.VMEM((BATCH,), jnp.float32),
        pltpu.VMEM_SHARED((NPAD, H), EDT),
        pltpu.VMEM_SHARED((NCNT,), jnp.float32),
    ] + [pltpu.SemaphoreType.DMA] * k

    def body(xn_hbm, e3_hbm, zagg_hbm, zcnt_hbm, *rest):
        if with_cnt:
            agg_out, cnt_out = rest[0], rest[1]
            rest = rest[2:]
        else:
            agg_out = rest[0]
            rest = rest[1:]
        idx_src, idx_dst, rows, ones_v = rest[:4]
        sh_agg, sh_cnt = rest[4], rest[5]
        sems = rest[6:6 + k]

        c = lax.axis_index("c")
        s = lax.axis_index("s")
        wid = c * NS + s
        base = wid * RPT + jnp.minimum(wid, REM)
        for r in (RPT, RPT + 1):
            for i in range(BATCH // 16):
                idx_src[r, pl.ds(i * 16, 16)] = jnp.zeros((16,), jnp.int32)
                idx_dst[r, pl.ds(i * 16, 16)] = jnp.full((16,), N, jnp.int32)
        pltpu.sync_copy(e3_hbm.at[0].at[pl.ds(base, RPT)],
                        idx_src.at[pl.ds(0, RPT)])
        pltpu.sync_copy(e3_hbm.at[1].at[pl.ds(base, RPT)],
                        idx_dst.at[pl.ds(0, RPT)])

        @pl.when(wid < REM)
        def _extra_row():
            pltpu.sync_copy(e3_hbm.at[0].at[pl.ds(base + RPT, 1)],
                            idx_src.at[pl.ds(RPT, 1)])
            pltpu.sync_copy(e3_hbm.at[1].at[pl.ds(base + RPT, 1)],
                            idx_dst.at[pl.ds(RPT, 1)])

        if with_cnt:
            for i in range(BATCH // 16):
                ones_v[pl.ds(i * 16, 16)] = jnp.ones((16,), jnp.float32)

        rz = pl.multiple_of(s * (NPAD // NS), NPAD // NS)
        pltpu.sync_copy(zagg_hbm.at[pl.ds(rz, NPAD // NS)],
                        sh_agg.at[pl.ds(rz, NPAD // NS)])
        if with_cnt:
            rc = pl.multiple_of(s * (NCNT // NS), NCNT // NS)
            pltpu.sync_copy(zcnt_hbm.at[pl.ds(rc, NCNT // NS)],
                            sh_cnt.at[pl.ds(rc, NCNT // NS)])

        plsc.subcore_barrier()

        for i in range(k):
            pltpu.async_copy(xn_hbm.at[idx_src.at[i]], rows.at[i], sems[i])

        def group(g, carry):
            for i in range(k):
                b = g * k + i
                pltpu.make_async_copy(xn_hbm.at[idx_src.at[b]], rows.at[i],
                                      sems[i]).wait()
                pltpu.sync_copy(rows.at[i], sh_agg.at[idx_dst.at[b]],
                                add=True)
                if with_cnt:
                    pltpu.sync_copy(ones_v, sh_cnt.at[idx_dst.at[b]],
                                    add=True)

                @pl.when(g < ngroups - 1)
                def _rearm():
                    pltpu.async_copy(xn_hbm.at[idx_src.at[b + k]],
                                     rows.at[i], sems[i])
            return carry

        lax.fori_loop(0, ngroups, group, 0)
        plsc.subcore_barrier()

        ro = pl.multiple_of(s * (N // NS), N // NS)
        pltpu.sync_copy(sh_agg.at[pl.ds(ro, N // NS)],
                        agg_out.at[c].at[pl.ds(ro, N // NS)])
        if with_cnt:
            rc2 = pl.multiple_of(s * (NCNT // NS), NCNT // NS)
            pltpu.sync_copy(sh_cnt.at[pl.ds(rc2, NCNT // NS)],
                            cnt_out.at[c].at[pl.ds(rc2, NCNT // NS)])

    fn = pl.kernel(body, out_type=out_type, mesh=mesh, scratch_types=scratch,
                   compiler_params=pltpu.CompilerParams(
                       use_tc_tiling_on_sc=False))
    return fn(xn, e3, zagg, zcnt)


def kernel(x, edge_index, W1_self, b1_self, W1_neigh, b1_neigh,
           W2_self, b2_self, W2_neigh, b2_neigh, W_out, b_out,
           W_lin1, b_lin1):
    e3 = edge_index.reshape(2, EROWS, BATCH)
    zagg = jnp.zeros((NPAD, H), EDT)
    zcnt = jnp.zeros((NCNT,), jnp.float32)

    def rep(b):
        return jnp.tile(b.reshape(1, -1), (8, 1))

    xn1, xs1 = _tc1(x, W1_neigh.T, W1_self.T, rep(b1_self))
    agg1, cnt = _sc_agg_call(xn1, e3, zagg, zcnt, with_cnt=True)
    cnt3 = cnt[:, :N].reshape(2, N, 1)
    xn2, xs2 = _tc2(agg1, cnt3, xs1, rep(b1_neigh), W2_neigh.T, W2_self.T,
                    rep(b2_self))

    agg2 = _sc_agg_call(xn2, e3, zagg, zcnt, with_cnt=False)
    h2, out = _tc3(agg2, cnt3, xs2, rep(b2_neigh), W_out.T, rep(b_out))
    return (h2, out)


# speedup vs baseline: 23.2734x; 1.3629x over previous
import functools

import jax
import jax.numpy as jnp
from jax import lax
from jax.experimental import pallas as pl
from jax.experimental.pallas import tpu as pltpu
from jax.experimental.pallas import tpu_sc as plsc

N = 10000
E = 320000
F = 128
H = 32
OUT = 128

NC = 2
NS = 16
NW = NC * NS
BATCH = 128
EROWS = E // BATCH
RPT = EROWS // NW
REM = EROWS - RPT * NW
TPW = 80
NPAD = N + 16
NCNT = 10240
BLK = 2000
GRID = N // BLK
EDT = jnp.bfloat16


def _mm1_body(x_ref, wn_ref, ws_ref, bs_ref, xn_ref, xs_ref):
    xb = x_ref[:]
    xn_ref[:] = jnp.dot(
        xb, wn_ref[:], preferred_element_type=jnp.float32).astype(EDT)
    xs_ref[:] = (jnp.dot(xb, ws_ref[:], preferred_element_type=jnp.float32)
                 + bs_ref[0:1, :])


def _tc1(x, wnT, wsT, bs):
    return pl.pallas_call(
        _mm1_body,
        grid=(GRID,),
        in_specs=[
            pl.BlockSpec((BLK, F), lambda i: (i, 0)),
            pl.BlockSpec((F, H), lambda i: (0, 0)),
            pl.BlockSpec((F, H), lambda i: (0, 0)),
            pl.BlockSpec((8, H), lambda i: (0, 0)),
        ],
        out_specs=[
            pl.BlockSpec((BLK, H), lambda i: (i, 0)),
            pl.BlockSpec((BLK, H), lambda i: (i, 0)),
        ],
        out_shape=[
            jax.ShapeDtypeStruct((N, H), EDT),
            jax.ShapeDtypeStruct((N, H), jnp.float32),
        ],
    )(x, wnT, wsT, bs)


def _mm2_body(agg_ref, cnt_ref, xs1_ref, bn_ref, w2n_ref, w2s_ref, b2s_ref,
              xn2_ref, xs2_ref):
    asum = (agg_ref[0].astype(jnp.float32)
            + agg_ref[1].astype(jnp.float32))
    csum = cnt_ref[0] + cnt_ref[1]
    mean = asum / jnp.maximum(csum, 1.0)
    h1 = jnp.maximum(mean + bn_ref[0:1, :] + xs1_ref[:], 0.0)
    xn2_ref[:] = jnp.dot(
        h1, w2n_ref[:], preferred_element_type=jnp.float32).astype(EDT)
    xs2_ref[:] = (jnp.dot(h1, w2s_ref[:], preferred_element_type=jnp.float32)
                  + b2s_ref[0:1, :])


def _tc2(agg, cnt3, xs1, bn, w2nT, w2sT, b2s):
    return pl.pallas_call(
        _mm2_body,
        grid=(GRID,),
        in_specs=[
            pl.BlockSpec((2, BLK, H), lambda i: (0, i, 0)),
            pl.BlockSpec((2, BLK, 1), lambda i: (0, i, 0)),
            pl.BlockSpec((BLK, H), lambda i: (i, 0)),
            pl.BlockSpec((8, H), lambda i: (0, 0)),
            pl.BlockSpec((H, H), lambda i: (0, 0)),
            pl.BlockSpec((H, H), lambda i: (0, 0)),
            pl.BlockSpec((8, H), lambda i: (0, 0)),
        ],
        out_specs=[
            pl.BlockSpec((BLK, H), lambda i: (i, 0)),
            pl.BlockSpec((BLK, H), lambda i: (i, 0)),
        ],
        out_shape=[
            jax.ShapeDtypeStruct((N, H), EDT),
            jax.ShapeDtypeStruct((N, H), jnp.float32),
        ],
    )(agg, cnt3, xs1, bn, w2nT, w2sT, b2s)


def _mm3_body(agg_ref, cnt_ref, xs2_ref, bn_ref, wout_ref, bout_ref,
              h2_ref, out_ref):
    asum = (agg_ref[0].astype(jnp.float32)
            + agg_ref[1].astype(jnp.float32))
    csum = cnt_ref[0] + cnt_ref[1]
    mean = asum / jnp.maximum(csum, 1.0)
    h2 = jnp.maximum(mean + bn_ref[0:1, :] + xs2_ref[:], 0.0)
    h2_ref[:] = h2
    out_ref[:] = (jnp.dot(h2, wout_ref[:], preferred_element_type=jnp.float32)
                  + bout_ref[0:1, :])


def _tc3(agg, cnt3, xs2, bn, woutT, bout):
    return pl.pallas_call(
        _mm3_body,
        grid=(GRID,),
        in_specs=[
            pl.BlockSpec((2, BLK, H), lambda i: (0, i, 0)),
            pl.BlockSpec((2, BLK, 1), lambda i: (0, i, 0)),
            pl.BlockSpec((BLK, H), lambda i: (i, 0)),
            pl.BlockSpec((8, H), lambda i: (0, 0)),
            pl.BlockSpec((H, OUT), lambda i: (0, 0)),
            pl.BlockSpec((8, OUT), lambda i: (0, 0)),
        ],
        out_specs=[
            pl.BlockSpec((BLK, H), lambda i: (i, 0)),
            pl.BlockSpec((BLK, OUT), lambda i: (i, 0)),
        ],
        out_shape=[
            jax.ShapeDtypeStruct((N, H), jnp.float32),
            jax.ShapeDtypeStruct((N, OUT), jnp.float32),
        ],
    )(agg, cnt3, xs2, bn, woutT, bout)


def _sc_agg_call(xn, e3, zagg, zcnt, with_cnt):
    mesh = plsc.VectorSubcoreMesh(core_axis_name="c", subcore_axis_name="s")
    if with_cnt:
        out_type = (jax.ShapeDtypeStruct((2, N, H), EDT),
                    jax.ShapeDtypeStruct((2, NCNT), jnp.float32))
    else:
        out_type = jax.ShapeDtypeStruct((2, N, H), EDT)

    k = 4 if with_cnt else 8
    ngroups = TPW // k

    scratch = [
        pltpu.VMEM((TPW, BATCH), jnp.int32),
        pltpu.VMEM((TPW, BATCH), jnp.int32),
        pltpu.VMEM((k, BATCH, H), EDT),
        pltpu.VMEM((BATCH,), jnp.float32),
        pltpu.VMEM_SHARED((NPAD, H), EDT),
        pltpu.VMEM_SHARED((NCNT,), jnp.float32),
        pltpu.VMEM_SHARED((N, H), EDT),
    ] + [pltpu.SemaphoreType.DMA] * k

    def body(xn_hbm, e3_hbm, zagg_hbm, zcnt_hbm, *rest):
        if with_cnt:
            agg_out, cnt_out = rest[0], rest[1]
            rest = rest[2:]
        else:
            agg_out = rest[0]
            rest = rest[1:]
        idx_src, idx_dst, rows, ones_v = rest[:4]
        sh_agg, sh_cnt, sh_xn = rest[4], rest[5], rest[6]
        sems = rest[7:7 + k]

        c = lax.axis_index("c")
        s = lax.axis_index("s")
        wid = c * NS + s
        base = wid * RPT + jnp.minimum(wid, REM)
        for r in (RPT, RPT + 1):
            for i in range(BATCH // 16):
                idx_src[r, pl.ds(i * 16, 16)] = jnp.zeros((16,), jnp.int32)
                idx_dst[r, pl.ds(i * 16, 16)] = jnp.full((16,), N, jnp.int32)
        pltpu.sync_copy(e3_hbm.at[0].at[pl.ds(base, RPT)],
                        idx_src.at[pl.ds(0, RPT)])
        pltpu.sync_copy(e3_hbm.at[1].at[pl.ds(base, RPT)],
                        idx_dst.at[pl.ds(0, RPT)])

        @pl.when(wid < REM)
        def _extra_row():
            pltpu.sync_copy(e3_hbm.at[0].at[pl.ds(base + RPT, 1)],
                            idx_src.at[pl.ds(RPT, 1)])
            pltpu.sync_copy(e3_hbm.at[1].at[pl.ds(base + RPT, 1)],
                            idx_dst.at[pl.ds(RPT, 1)])

        if with_cnt:
            for i in range(BATCH // 16):
                ones_v[pl.ds(i * 16, 16)] = jnp.ones((16,), jnp.float32)

        rz = pl.multiple_of(s * (NPAD // NS), NPAD // NS)
        pltpu.sync_copy(zagg_hbm.at[pl.ds(rz, NPAD // NS)],
                        sh_agg.at[pl.ds(rz, NPAD // NS)])
        rx = pl.multiple_of(s * (N // NS), N // NS)
        pltpu.sync_copy(xn_hbm.at[pl.ds(rx, N // NS)],
                        sh_xn.at[pl.ds(rx, N // NS)])
        if with_cnt:
            rc = pl.multiple_of(s * (NCNT // NS), NCNT // NS)
            pltpu.sync_copy(zcnt_hbm.at[pl.ds(rc, NCNT // NS)],
                            sh_cnt.at[pl.ds(rc, NCNT // NS)])

        plsc.subcore_barrier()

        for i in range(k):
            pltpu.async_copy(sh_xn.at[idx_src.at[i]], rows.at[i], sems[i])

        def group(g, carry):
            for i in range(k):
                b = g * k + i
                pltpu.make_async_copy(sh_xn.at[idx_src.at[b]], rows.at[i],
                                      sems[i]).wait()
                pltpu.sync_copy(rows.at[i], sh_agg.at[idx_dst.at[b]],
                                add=True)
                if with_cnt:
                    pltpu.sync_copy(ones_v, sh_cnt.at[idx_dst.at[b]],
                                    add=True)

                @pl.when(g < ngroups - 1)
                def _rearm():
                    pltpu.async_copy(sh_xn.at[idx_src.at[b + k]],
                                     rows.at[i], sems[i])
            return carry

        lax.fori_loop(0, ngroups, group, 0)
        plsc.subcore_barrier()

        ro = pl.multiple_of(s * (N // NS), N // NS)
        pltpu.sync_copy(sh_agg.at[pl.ds(ro, N // NS)],
                        agg_out.at[c].at[pl.ds(ro, N // NS)])
        if with_cnt:
            rc2 = pl.multiple_of(s * (NCNT // NS), NCNT // NS)
            pltpu.sync_copy(sh_cnt.at[pl.ds(rc2, NCNT // NS)],
                            cnt_out.at[c].at[pl.ds(rc2, NCNT // NS)])

    fn = pl.kernel(body, out_type=out_type, mesh=mesh, scratch_types=scratch,
                   compiler_params=pltpu.CompilerParams(
                       use_tc_tiling_on_sc=False))
    return fn(xn, e3, zagg, zcnt)


def kernel(x, edge_index, W1_self, b1_self, W1_neigh, b1_neigh,
           W2_self, b2_self, W2_neigh, b2_neigh, W_out, b_out,
           W_lin1, b_lin1):
    e3 = edge_index.reshape(2, EROWS, BATCH)
    zagg = jnp.zeros((NPAD, H), EDT)
    zcnt = jnp.zeros((NCNT,), jnp.float32)

    def rep(b):
        return jnp.tile(b.reshape(1, -1), (8, 1))

    xn1, xs1 = _tc1(x, W1_neigh.T, W1_self.T, rep(b1_self))
    agg1, cnt = _sc_agg_call(xn1, e3, zagg, zcnt, with_cnt=True)
    cnt3 = cnt[:, :N].reshape(2, N, 1)
    xn2, xs2 = _tc2(agg1, cnt3, xs1, rep(b1_neigh), W2_neigh.T, W2_self.T,
                    rep(b2_self))

    agg2 = _sc_agg_call(xn2, e3, zagg, zcnt, with_cnt=False)
    h2, out = _tc3(agg2, cnt3, xs2, rep(b2_neigh), W_out.T, rep(b_out))
    return (h2, out)
